# baseline (speedup 1.0000x reference)
import functools

import jax
import jax.numpy as jnp
from jax import lax
from jax.experimental import pallas as pl
from jax.experimental.pallas import tpu as pltpu
from jax.experimental.pallas import tpu_sc as plsc

N_NODES_K = 10000
N_EDGES_K = 320000
NC = 2
NS = 16
NW = NC * NS
LANES = 16
EPW = N_EDGES_K // NW
HPAD = 10240


def _hist_body(recv_hbm, parts_hbm, idx_v, hist_v):
  c = lax.axis_index("c")
  s = lax.axis_index("s")
  wid = s * NC + c

  pltpu.sync_copy(recv_hbm.at[pl.ds(wid * EPW, EPW)], idx_v)

  def zero(i, carry):
    hist_v[pl.ds(i * LANES, LANES)] = jnp.zeros((LANES,), jnp.float32)
    return carry

  lax.fori_loop(0, HPAD // LANES, zero, 0)

  def body(i, carry):
    v = idx_v[pl.ds(i * LANES, LANES)]
    cnt, last = plsc.scan_count(v)
    plsc.addupdate_scatter(hist_v, [v], cnt.astype(jnp.float32), mask=last)
    return carry

  lax.fori_loop(0, EPW // LANES, body, 0)

  pltpu.sync_copy(hist_v, parts_hbm.at[wid])


_hist = pl.kernel(
    _hist_body,
    out_type=jax.ShapeDtypeStruct((NW, HPAD), jnp.float32),
    mesh=plsc.VectorSubcoreMesh(
        core_axis_name="c", subcore_axis_name="s", num_cores=NC,
        num_subcores=NS),
    scratch_types=[
        pltpu.VMEM((EPW,), jnp.int32),
        pltpu.VMEM((HPAD,), jnp.float32),
    ],
    compiler_params=pltpu.CompilerParams(needs_layout_passes=False),
)


def _merge_body(parts_ref, out_ref):
  out_ref[...] = jnp.sum(parts_ref[...], axis=0, keepdims=True)


_merge = pl.pallas_call(
    _merge_body,
    out_shape=jax.ShapeDtypeStruct((1, HPAD), jnp.float32),
)


@jax.jit
def kernel(x, edge_index):
  n = x.shape[0]
  recv = edge_index[1].astype(jnp.int32)
  parts = _hist(recv)
  merged = _merge(parts)
  return merged[0, :n].reshape(n, 1)
